# baseline (speedup 1.0000x reference)
import functools

import jax
import jax.numpy as jnp
from jax import lax
from jax.experimental import pallas as pl
from jax.experimental.pallas import tpu as pltpu
from jax.experimental.pallas import tpu_sc as plsc

N = 10000
E = 320000
B = 8
D = 128
VOCAB = 100000

NC = 2
NS = 16
NW = NC * NS

NPAD = 10240
ROWS_PER_W = NPAD // NW
GCH = 80
NGCH = ROWS_PER_W // GCH

SCH = 128
NSCH = 80
NBUF = 2
NH = 2
CPH = NSCH // NH
EPW = NSCH * SCH
EPAD = NW * EPW
ROWS_PER_TILE = NPAD // NS

_SC_CACHE = {}


def _get_sc_gather():
    if "gather" not in _SC_CACHE:
        mesh = plsc.VectorSubcoreMesh(core_axis_name="c", subcore_axis_name="s")
        _SC_CACHE["gather"] = functools.partial(
            pl.kernel,
            mesh=mesh,
            out_type=jax.ShapeDtypeStruct((NPAD, D), jnp.float32),
            scratch_types=[
                pltpu.VMEM((NGCH, GCH), jnp.int32),
                pltpu.VMEM((ROWS_PER_W, D), jnp.float32),
                pltpu.SemaphoreType.DMA,
            ],
        )(_sc_gather_body)
    return _SC_CACHE["gather"]


def _sc_gather_body(table_hbm, ids_hbm, out_hbm, idx_v, rows_v, sem):
    cid = lax.axis_index("c")
    sid = lax.axis_index("s")
    wid = sid * NC + cid
    pltpu.sync_copy(ids_hbm.at[wid], idx_v)
    copies = []
    for j in range(NGCH):
        copies.append(
            pltpu.async_copy(
                table_hbm.at[idx_v.at[j]],
                rows_v.at[pl.ds(j * GCH, GCH)],
                sem,
            )
        )
    for c in copies:
        c.wait()
    pltpu.sync_copy(rows_v, out_hbm.at[pl.ds(wid * ROWS_PER_W, ROWS_PER_W)])


def _get_sc_scatter():
    if "scatter" not in _SC_CACHE:
        mesh = plsc.VectorSubcoreMesh(core_axis_name="c", subcore_axis_name="s")
        _SC_CACHE["scatter"] = functools.partial(
            pl.kernel,
            mesh=mesh,
            out_type=jax.ShapeDtypeStruct((NC, NPAD, D), jnp.float32),
            scratch_types=[
                pltpu.VMEM((CPH, SCH), jnp.int32),
                pltpu.VMEM((CPH, SCH), jnp.int32),
                pltpu.VMEM((NBUF, SCH, D), jnp.float32),
                pltpu.VMEM_SHARED((NPAD, D), jnp.float32),
                pltpu.SemaphoreType.DMA,
                pltpu.SemaphoreType.DMA,
                pltpu.SemaphoreType.DMA,
                pltpu.SemaphoreType.DMA,
            ],
        )(_sc_scatter_body)
    return _SC_CACHE["scatter"]


def _sc_scatter_body(x0_hbm, src_hbm, dst_hbm, zeros_hbm, out_hbm,
                     src_v, dst_v, rows_v, agg_sh, g0, g1, s0, s1):
    cid = lax.axis_index("c")
    sid = lax.axis_index("s")
    row0 = sid * ROWS_PER_TILE
    pltpu.sync_copy(zeros_hbm, agg_sh.at[pl.ds(row0, ROWS_PER_TILE)])
    plsc.subcore_barrier()

    gsem = [g0, g1]
    ssem = [s0, s1]
    for h in range(NH):
        pltpu.sync_copy(src_hbm.at[cid, sid, pl.ds(h * CPH, CPH)], src_v)
        pltpu.sync_copy(dst_hbm.at[cid, sid, pl.ds(h * CPH, CPH)], dst_v)
        for b in range(NBUF):
            pltpu.async_copy(x0_hbm.at[src_v.at[b]], rows_v.at[b], gsem[b])

        def group(i, carry):
            j = i * NBUF
            for b in range(NBUF):
                pltpu.make_async_copy(
                    x0_hbm.at[src_v.at[0]], rows_v.at[b], gsem[b]
                ).wait()
                pltpu.sync_copy(rows_v.at[b], agg_sh.at[dst_v.at[j + b]],
                                add=True)
                pltpu.async_copy(
                    x0_hbm.at[src_v.at[j + b + NBUF]], rows_v.at[b], gsem[b]
                )
            return carry

        lax.fori_loop(0, CPH // NBUF - 1, group, 0)
        jlast = CPH - NBUF
        for b in range(NBUF):
            pltpu.make_async_copy(
                x0_hbm.at[src_v.at[0]], rows_v.at[b], gsem[b]
            ).wait()
            pltpu.sync_copy(rows_v.at[b], agg_sh.at[dst_v.at[jlast + b]],
                            add=True)
    plsc.subcore_barrier()
    pltpu.sync_copy(
        agg_sh.at[pl.ds(row0, ROWS_PER_TILE)],
        out_hbm.at[cid, pl.ds(row0, ROWS_PER_TILE)],
    )


_VBLK = 2048
_VN = -(-VOCAB // _VBLK)
_LBLK = 1280
_LN = NPAD // _LBLK


def _tc1_body(emb_ref, linw_ref, linb_ref, e_ref, w_ref,
              x0_ref, acc_ref, rs_ref):
    i = pl.program_id(0)

    @pl.when(i < _LN)
    def _():
        x0_ref[...] = (
            lax.dot_general(emb_ref[...], linw_ref[...],
                            (((1,), (1,)), ((), ())),
                            preferred_element_type=jnp.float32)
            + linb_ref[...]
        )

    valid = VOCAB - i * _VBLK
    col = lax.broadcasted_iota(jnp.int32, (1, _VBLK), 1)
    row = lax.broadcasted_iota(jnp.int32, (_VBLK, 1), 0)
    e = jnp.where(col < valid, e_ref[...], 0.0)
    w = jnp.where(row < valid, w_ref[...], 0.0)
    pacc = jnp.dot(e, w, preferred_element_type=jnp.float32)
    prs = jnp.sum(e, axis=1, keepdims=True)

    @pl.when(i == 0)
    def _():
        acc_ref[...] = jnp.zeros_like(acc_ref)
        rs_ref[...] = jnp.zeros_like(rs_ref)

    acc_ref[...] += pacc
    rs_ref[...] += jnp.broadcast_to(prs, (B, D))


def _tc1_call(emb, lin_W, lin_b2, ehr, table):
    lidx = lambda i: (jnp.minimum(i, _LN - 1), 0)
    return pl.pallas_call(
        _tc1_body,
        grid=(_VN,),
        in_specs=[
            pl.BlockSpec((_LBLK, D), lidx),
            pl.BlockSpec((D, D), lambda i: (0, 0)),
            pl.BlockSpec((1, D), lambda i: (0, 0)),
            pl.BlockSpec((B, _VBLK), lambda i: (0, i)),
            pl.BlockSpec((_VBLK, D), lambda i: (i, 0)),
        ],
        out_specs=[
            pl.BlockSpec((_LBLK, D), lidx),
            pl.BlockSpec((B, D), lambda i: (0, 0)),
            pl.BlockSpec((B, D), lambda i: (0, 0)),
        ],
        out_shape=[
            jax.ShapeDtypeStruct((NPAD, D), jnp.float32),
            jax.ShapeDtypeStruct((B, D), jnp.float32),
            jax.ShapeDtypeStruct((B, D), jnp.float32),
        ],
    )(emb, lin_W, lin_b2, ehr, table)


_GBLK = 2048


def _tc2_body(x_ref, agg_ref, bt_ref, w_ref, b_ref, acc_ref, rs_ref,
              linw_ref, linb_ref, mlpw_ref, mlpb_ref, o_ref,
              sums_s, cnts_s):
    i = pl.program_id(0)
    z = x_ref[...] + agg_ref[0] + agg_ref[1]
    h = (
        lax.dot_general(z, w_ref[...], (((1,), (1,)), ((), ())),
                        preferred_element_type=jnp.float32)
        + b_ref[...]
    )
    h = jnp.maximum(h, 0.0)
    bt = bt_ref[0, 0, :]
    oh = (bt[:, None] == lax.broadcasted_iota(jnp.int32, (1, B), 1)
          ).astype(jnp.float32)
    psums = lax.dot_general(oh, h, (((0,), (0,)), ((), ())),
                            preferred_element_type=jnp.float32)
    pcnts = jnp.sum(oh, axis=0)

    @pl.when(i == 0)
    def _():
        sums_s[...] = jnp.zeros_like(sums_s)
        cnts_s[...] = jnp.zeros_like(cnts_s)

    sums_s[...] += psums
    cnts_s[...] += jnp.broadcast_to(pcnts[:, None], (B, D))

    @pl.when(i == NPAD // _GBLK - 1)
    def _():
        xg = sums_s[...] / jnp.maximum(cnts_s[...], 1.0)
        xn = (
            lax.dot_general(acc_ref[...] / rs_ref[...], linw_ref[...],
                            (((1,), (1,)), ((), ())),
                            preferred_element_type=jnp.float32)
            + linb_ref[...]
        )
        wg = mlpw_ref[:, :D]
        wn = mlpw_ref[:, D:]
        o_ref[...] = (
            lax.dot_general(xg, wg, (((1,), (1,)), ((), ())),
                            preferred_element_type=jnp.float32)
            + lax.dot_general(xn, wn, (((1,), (1,)), ((), ())),
                              preferred_element_type=jnp.float32)
            + mlpb_ref[...]
        )


def _tc2_call(x0, aggp, bt_resh, w, b2, acc, rs, lin_W, lin_b2,
              mlp_W, mlp_b2):
    nblk = NPAD // _GBLK
    full = lambda i: (0, 0)
    return pl.pallas_call(
        _tc2_body,
        grid=(nblk,),
        in_specs=[
            pl.BlockSpec((_GBLK, D), lambda i: (i, 0)),
            pl.BlockSpec((NC, _GBLK, D), lambda i: (0, i, 0)),
            pl.BlockSpec((1, 1, _GBLK), lambda i: (i, 0, 0)),
            pl.BlockSpec((D, D), full),
            pl.BlockSpec((1, D), full),
            pl.BlockSpec((B, D), full),
            pl.BlockSpec((B, D), full),
            pl.BlockSpec((D, D), full),
            pl.BlockSpec((1, D), full),
            pl.BlockSpec((D, 2 * D), full),
            pl.BlockSpec((1, D), full),
        ],
        out_specs=pl.BlockSpec((B, D), full),
        out_shape=jax.ShapeDtypeStruct((B, D), jnp.float32),
        scratch_shapes=[
            pltpu.VMEM((B, D), jnp.float32),
            pltpu.VMEM((B, D), jnp.float32),
        ],
    )(x0, aggp, bt_resh, w, b2, acc, rs, lin_W, lin_b2, mlp_W, mlp_b2)


def kernel(node_ids, edge_ids, edge_index, edge_attr, visit_times,
           visit_order, visit_node, ehr_nodes, batch, attn_mask,
           node_emb_W, edge_emb_W, lin_W, lin_b, beta_W, beta_b,
           gin_W, gin_b, mlp_W, mlp_b):
    node_ids = node_ids.astype(jnp.int32)
    ids_pad = jnp.concatenate(
        [node_ids, jnp.zeros((NPAD - N,), jnp.int32)]
    ).reshape(NW, NGCH, GCH)
    emb = _get_sc_gather()(node_emb_W, ids_pad)

    lin_b2 = lin_b.reshape(1, D)
    x0, acc, rs = _tc1_call(emb, lin_W, lin_b2, ehr_nodes, node_emb_W)

    src = edge_index[0].astype(jnp.int32)
    dst = edge_index[1].astype(jnp.int32)
    npad_e = EPAD - E
    pad_idx = N + (jnp.arange(npad_e, dtype=jnp.int32) % (NPAD - N))
    src_slab = jnp.concatenate([src, pad_idx]).reshape(NC, NS, NSCH, SCH)
    dst_slab = jnp.concatenate([dst, pad_idx]).reshape(NC, NS, NSCH, SCH)
    zeros_hbm = jnp.zeros((ROWS_PER_TILE, D), jnp.float32)
    aggp = _get_sc_scatter()(x0, src_slab, dst_slab, zeros_hbm)

    bt_pad = jnp.concatenate(
        [batch.astype(jnp.int32), jnp.full((NPAD - N,), B, jnp.int32)]
    ).reshape(NPAD // _GBLK, 1, _GBLK)
    gin_b2 = gin_b.reshape(1, D)
    mlp_b2 = mlp_b.reshape(1, D)
    logits = _tc2_call(x0, aggp, bt_pad, gin_W, gin_b2, acc, rs,
                       lin_W, lin_b2, mlp_W, mlp_b2)
    return logits


# speedup vs baseline: 1.1945x; 1.1945x over previous
import functools

import jax
import jax.numpy as jnp
from jax import lax
from jax.experimental import pallas as pl
from jax.experimental.pallas import tpu as pltpu
from jax.experimental.pallas import tpu_sc as plsc

N = 10000
E = 320000
B = 8
D = 128
VOCAB = 100000

NC = 2
NS = 16
NW = NC * NS

NPAD = 10240
ROWS_PER_W = NPAD // NW
GCH = 80
NGCH = ROWS_PER_W // GCH

SCH = 64
NSCH = 160
NBUF = 4
NH = 4
CPH = NSCH // NH
EPW = NSCH * SCH
EPAD = NW * EPW
ROWS_PER_TILE = NPAD // NS

_SC_CACHE = {}


def _get_sc_gather():
    if "gather" not in _SC_CACHE:
        mesh = plsc.VectorSubcoreMesh(core_axis_name="c", subcore_axis_name="s")
        _SC_CACHE["gather"] = functools.partial(
            pl.kernel,
            mesh=mesh,
            out_type=jax.ShapeDtypeStruct((NPAD, D), jnp.float32),
            scratch_types=[
                pltpu.VMEM((NGCH, GCH), jnp.int32),
                pltpu.VMEM((ROWS_PER_W, D), jnp.float32),
                pltpu.SemaphoreType.DMA,
            ],
        )(_sc_gather_body)
    return _SC_CACHE["gather"]


def _sc_gather_body(table_hbm, ids_hbm, out_hbm, idx_v, rows_v, sem):
    cid = lax.axis_index("c")
    sid = lax.axis_index("s")
    wid = sid * NC + cid
    pltpu.sync_copy(ids_hbm.at[wid], idx_v)
    copies = []
    for j in range(NGCH):
        copies.append(
            pltpu.async_copy(
                table_hbm.at[idx_v.at[j]],
                rows_v.at[pl.ds(j * GCH, GCH)],
                sem,
            )
        )
    for c in copies:
        c.wait()
    pltpu.sync_copy(rows_v, out_hbm.at[pl.ds(wid * ROWS_PER_W, ROWS_PER_W)])


def _get_sc_scatter():
    if "scatter" not in _SC_CACHE:
        mesh = plsc.VectorSubcoreMesh(core_axis_name="c", subcore_axis_name="s")
        _SC_CACHE["scatter"] = functools.partial(
            pl.kernel,
            mesh=mesh,
            out_type=jax.ShapeDtypeStruct((NC, NPAD, D), jnp.float32),
            scratch_types=[
                pltpu.VMEM((CPH, SCH), jnp.int32),
                pltpu.VMEM((CPH, SCH), jnp.int32),
                pltpu.VMEM((NBUF, SCH, D), jnp.float32),
                pltpu.VMEM_SHARED((NPAD, D), jnp.float32),
                pltpu.SemaphoreType.DMA,
                pltpu.SemaphoreType.DMA,
                pltpu.SemaphoreType.DMA,
                pltpu.SemaphoreType.DMA,
            ],
        )(_sc_scatter_body)
    return _SC_CACHE["scatter"]


def _sc_scatter_body(x0_hbm, src_hbm, dst_hbm, zeros_hbm, out_hbm,
                     src_v, dst_v, rows_v, agg_sh, g0, g1, s0, s1):
    cid = lax.axis_index("c")
    sid = lax.axis_index("s")
    row0 = sid * ROWS_PER_TILE
    pltpu.sync_copy(zeros_hbm, agg_sh.at[pl.ds(row0, ROWS_PER_TILE)])
    plsc.subcore_barrier()

    gsem = [g0, g1, s0, s1][:NBUF]
    for h in range(NH):
        pltpu.sync_copy(src_hbm.at[cid, sid, pl.ds(h * CPH, CPH)], src_v)
        pltpu.sync_copy(dst_hbm.at[cid, sid, pl.ds(h * CPH, CPH)], dst_v)
        for b in range(NBUF):
            pltpu.async_copy(x0_hbm.at[src_v.at[b]], rows_v.at[b], gsem[b])

        def group(i, carry):
            j = i * NBUF
            for b in range(NBUF):
                pltpu.make_async_copy(
                    x0_hbm.at[src_v.at[0]], rows_v.at[b], gsem[b]
                ).wait()
                pltpu.sync_copy(rows_v.at[b], agg_sh.at[dst_v.at[j + b]],
                                add=True)
                pltpu.async_copy(
                    x0_hbm.at[src_v.at[j + b + NBUF]], rows_v.at[b], gsem[b]
                )
            return carry

        lax.fori_loop(0, CPH // NBUF - 1, group, 0)
        jlast = CPH - NBUF
        for b in range(NBUF):
            pltpu.make_async_copy(
                x0_hbm.at[src_v.at[0]], rows_v.at[b], gsem[b]
            ).wait()
            pltpu.sync_copy(rows_v.at[b], agg_sh.at[dst_v.at[jlast + b]],
                            add=True)
    plsc.subcore_barrier()
    pltpu.sync_copy(
        agg_sh.at[pl.ds(row0, ROWS_PER_TILE)],
        out_hbm.at[cid, pl.ds(row0, ROWS_PER_TILE)],
    )


def _lin_body(x_ref, w_ref, b_ref, o_ref):
    o_ref[...] = (
        lax.dot_general(x_ref[...], w_ref[...], (((1,), (1,)), ((), ())),
                        preferred_element_type=jnp.float32)
        + b_ref[...]
    )


def _lin_call(x, w, b2):
    blk = 1280
    return pl.pallas_call(
        _lin_body,
        grid=(NPAD // blk,),
        in_specs=[
            pl.BlockSpec((blk, D), lambda i: (i, 0)),
            pl.BlockSpec((D, D), lambda i: (0, 0)),
            pl.BlockSpec((1, D), lambda i: (0, 0)),
        ],
        out_specs=pl.BlockSpec((blk, D), lambda i: (i, 0)),
        out_shape=jax.ShapeDtypeStruct((NPAD, D), jnp.float32),
    )(x, w, b2)


_GBLK = 2048


def _gin_body(x_ref, agg_ref, bt_ref, w_ref, b_ref, sums_ref, cnts_ref):
    z = x_ref[...] + agg_ref[0] + agg_ref[1]
    h = (
        lax.dot_general(z, w_ref[...], (((1,), (1,)), ((), ())),
                        preferred_element_type=jnp.float32)
        + b_ref[...]
    )
    h = jnp.maximum(h, 0.0)
    bt = bt_ref[0, 0, :]
    oh = (bt[:, None] == lax.broadcasted_iota(jnp.int32, (1, B), 1)
          ).astype(jnp.float32)
    psums = lax.dot_general(oh, h, (((0,), (0,)), ((), ())),
                            preferred_element_type=jnp.float32)
    pcnts = jnp.sum(oh, axis=0)

    @pl.when(pl.program_id(0) == 0)
    def _():
        sums_ref[...] = jnp.zeros_like(sums_ref)
        cnts_ref[...] = jnp.zeros_like(cnts_ref)

    sums_ref[...] += psums
    cnts_ref[...] += jnp.broadcast_to(pcnts[:, None], (B, D))


def _gin_call(x0, aggp, bt_resh, w, b2):
    nblk = NPAD // _GBLK
    return pl.pallas_call(
        _gin_body,
        grid=(nblk,),
        in_specs=[
            pl.BlockSpec((_GBLK, D), lambda i: (i, 0)),
            pl.BlockSpec((NC, _GBLK, D), lambda i: (0, i, 0)),
            pl.BlockSpec((1, 1, _GBLK), lambda i: (i, 0, 0)),
            pl.BlockSpec((D, D), lambda i: (0, 0)),
            pl.BlockSpec((1, D), lambda i: (0, 0)),
        ],
        out_specs=[
            pl.BlockSpec((B, D), lambda i: (0, 0)),
            pl.BlockSpec((B, D), lambda i: (0, 0)),
        ],
        out_shape=[
            jax.ShapeDtypeStruct((B, D), jnp.float32),
            jax.ShapeDtypeStruct((B, D), jnp.float32),
        ],
    )(x0, aggp, bt_resh, w, b2)


_VBLK = 2048
_VN = -(-VOCAB // _VBLK)


def _ehr_body(e_ref, w_ref, acc_ref, rs_ref):
    i = pl.program_id(0)
    valid = VOCAB - i * _VBLK
    col = lax.broadcasted_iota(jnp.int32, (1, _VBLK), 1)
    row = lax.broadcasted_iota(jnp.int32, (_VBLK, 1), 0)
    e = jnp.where(col < valid, e_ref[...], 0.0)
    w = jnp.where(row < valid, w_ref[...], 0.0)
    pacc = jnp.dot(e, w, preferred_element_type=jnp.float32)
    prs = jnp.sum(e, axis=1, keepdims=True)

    @pl.when(i == 0)
    def _():
        acc_ref[...] = jnp.zeros_like(acc_ref)
        rs_ref[...] = jnp.zeros_like(rs_ref)

    acc_ref[...] += pacc
    rs_ref[...] += jnp.broadcast_to(prs, (B, D))


def _ehr_call(ehr, table):
    return pl.pallas_call(
        _ehr_body,
        grid=(_VN,),
        in_specs=[
            pl.BlockSpec((B, _VBLK), lambda i: (0, i)),
            pl.BlockSpec((_VBLK, D), lambda i: (i, 0)),
        ],
        out_specs=[
            pl.BlockSpec((B, D), lambda i: (0, 0)),
            pl.BlockSpec((B, D), lambda i: (0, 0)),
        ],
        out_shape=[
            jax.ShapeDtypeStruct((B, D), jnp.float32),
            jax.ShapeDtypeStruct((B, D), jnp.float32),
        ],
    )(ehr, table)


def _combine_body(sums_ref, cnts_ref, acc_ref, rs_ref, linw_ref, linb_ref,
                  mlpw_ref, mlpb_ref, o_ref):
    xg = sums_ref[...] / jnp.maximum(cnts_ref[...], 1.0)
    xn = (
        lax.dot_general(acc_ref[...] / rs_ref[...], linw_ref[...],
                        (((1,), (1,)), ((), ())),
                        preferred_element_type=jnp.float32)
        + linb_ref[...]
    )
    wg = mlpw_ref[:, :D]
    wn = mlpw_ref[:, D:]
    o_ref[...] = (
        lax.dot_general(xg, wg, (((1,), (1,)), ((), ())),
                        preferred_element_type=jnp.float32)
        + lax.dot_general(xn, wn, (((1,), (1,)), ((), ())),
                          preferred_element_type=jnp.float32)
        + mlpb_ref[...]
    )


def _combine_call(sums, cnts, acc, rs, lin_W, lin_b2, mlp_W, mlp_b2):
    return pl.pallas_call(
        _combine_body,
        out_shape=jax.ShapeDtypeStruct((B, D), jnp.float32),
    )(sums, cnts, acc, rs, lin_W, lin_b2, mlp_W, mlp_b2)


def kernel(node_ids, edge_ids, edge_index, edge_attr, visit_times,
           visit_order, visit_node, ehr_nodes, batch, attn_mask,
           node_emb_W, edge_emb_W, lin_W, lin_b, beta_W, beta_b,
           gin_W, gin_b, mlp_W, mlp_b):
    node_ids = node_ids.astype(jnp.int32)
    ids_pad = jnp.concatenate(
        [node_ids, jnp.zeros((NPAD - N,), jnp.int32)]
    ).reshape(NW, NGCH, GCH)
    emb = _get_sc_gather()(node_emb_W, ids_pad)

    lin_b2 = lin_b.reshape(1, D)
    x0 = _lin_call(emb, lin_W, lin_b2)

    src = edge_index[0].astype(jnp.int32)
    dst = edge_index[1].astype(jnp.int32)
    npad_e = EPAD - E
    pad_idx = N + (jnp.arange(npad_e, dtype=jnp.int32) % (NPAD - N))
    src_slab = jnp.concatenate([src, pad_idx]).reshape(NC, NS, NSCH, SCH)
    dst_slab = jnp.concatenate([dst, pad_idx]).reshape(NC, NS, NSCH, SCH)
    zeros_hbm = jnp.zeros((ROWS_PER_TILE, D), jnp.float32)
    aggp = _get_sc_scatter()(x0, src_slab, dst_slab, zeros_hbm)

    acc, rs = _ehr_call(ehr_nodes, node_emb_W)

    bt_pad = jnp.concatenate(
        [batch.astype(jnp.int32), jnp.full((NPAD - N,), B, jnp.int32)]
    ).reshape(NPAD // _GBLK, 1, _GBLK)
    gin_b2 = gin_b.reshape(1, D)
    sums, cnts = _gin_call(x0, aggp, bt_pad, gin_W, gin_b2)

    mlp_b2 = mlp_b.reshape(1, D)
    logits = _combine_call(sums, cnts, acc, rs, lin_W, lin_b2, mlp_W, mlp_b2)
    return logits
